# Initial kernel scaffold; baseline (speedup 1.0000x reference)
import jax
import jax.numpy as jnp
from jax.experimental import pallas as pl


def kernel(x, table, pe):
    raise NotImplementedError("write your pallas kernel here")


# speedup vs baseline: 3.6609x; 3.6609x over previous
import math

import jax
import jax.numpy as jnp
from jax import lax
from jax.experimental import pallas as pl
from jax.experimental.pallas import tpu as pltpu, tpu_sc as plsc

N_TOKENS = 100000
D = 128
B = 1024
S = 200

NC = 2
NS = 16
NW = NC * NS
L = 16

CHUNK = 128
NCH = B // CHUNK
SCALE = math.sqrt(float(D))

S_BASE = S // NW
S_EXTRA = S % NW


def _sc_body(table_hbm, idx_hbm, pe_hbm, out_hbm, idx_v, rows_v, pe_v, sem):
    wid = lax.axis_index("s") * NC + lax.axis_index("c")
    n_s = S_BASE + (wid < S_EXTRA).astype(jnp.int32)
    start = wid * S_BASE + jnp.minimum(wid, S_EXTRA)

    def seq_body(li, carry):
        s = start + li
        pltpu.sync_copy(pe_hbm.at[s], pe_v)
        pe_vs = [pe_v[pl.ds(L * j, L)] for j in range(D // L)]

        def chunk_body(c, carry2):
            pltpu.sync_copy(idx_hbm.at[s, pl.ds(c * CHUNK, CHUNK)], idx_v)
            pltpu.async_copy(table_hbm.at[idx_v], rows_v, sem).wait()

            def row_body(r, carry3):
                for j in range(D // L):
                    v = rows_v[r, pl.ds(L * j, L)]
                    rows_v[r, pl.ds(L * j, L)] = v * SCALE + pe_vs[j]
                return carry3

            lax.fori_loop(0, CHUNK, row_body, 0)
            pltpu.sync_copy(rows_v, out_hbm.at[s, pl.ds(c * CHUNK, CHUNK)])
            return carry2

        lax.fori_loop(0, NCH, chunk_body, 0)
        return carry

    lax.fori_loop(0, n_s, seq_body, 0)


def kernel(x, table, pe):
    idx_t = jnp.transpose(x).astype(jnp.int32)
    pe2 = pe[:S, 0, :]

    mesh = plsc.VectorSubcoreMesh(
        core_axis_name="c", subcore_axis_name="s",
        num_cores=NC, num_subcores=NS,
    )
    out = pl.kernel(
        _sc_body,
        out_type=jax.ShapeDtypeStruct((S, B, D), jnp.float32),
        mesh=mesh,
        scratch_types=[
            pltpu.VMEM((CHUNK,), jnp.int32),
            pltpu.VMEM((CHUNK, D), jnp.float32),
            pltpu.VMEM((D,), jnp.float32),
            pltpu.SemaphoreType.DMA,
        ],
    )(table, idx_t, pe2)
    return out


# speedup vs baseline: 6.0459x; 1.6515x over previous
import math

import jax
import jax.numpy as jnp
from jax import lax
from jax.experimental import pallas as pl
from jax.experimental.pallas import tpu as pltpu, tpu_sc as plsc

N_TOKENS = 100000
D = 128
B = 1024
S = 200

NC = 2
NS = 16
NW = NC * NS
L = 16

CHUNK = 128
NCH = B // CHUNK
SCALE = math.sqrt(float(D))

S_BASE = S // NW
S_EXTRA = S % NW


def _sc_body(table_hbm, idx_hbm, pe_hbm, out_hbm,
             idx_v, rows0, rows1, pe_v, g0, g1, s0, s1):
    wid = lax.axis_index("s") * NC + lax.axis_index("c")
    n_s = S_BASE + (wid < S_EXTRA).astype(jnp.int32)
    start = wid * S_BASE + jnp.minimum(wid, S_EXTRA)

    bufs = (rows0, rows1)
    gsem = (g0, g1)
    ssem = (s0, s1)

    def seq_body(li, carry):
        s = start + li
        pltpu.sync_copy(pe_hbm.at[s], pe_v)
        pltpu.sync_copy(idx_hbm.at[s], idx_v)
        pe_vs = [pe_v[pl.ds(L * j, L)] for j in range(D // L)]

        g = [None, None]
        st = [None, None]
        g[0] = pltpu.async_copy(table_hbm.at[idx_v.at[0]], bufs[0], gsem[0])
        for c in range(NCH):
            p = c % 2
            q = (c + 1) % 2
            if c + 1 < NCH:
                if c >= 1:
                    st[q].wait()
                g[q] = pltpu.async_copy(
                    table_hbm.at[idx_v.at[c + 1]], bufs[q], gsem[q])
            g[p].wait()

            def row_body(r, carry3, _buf=bufs[p]):
                for u in range(2):
                    for j in range(D // L):
                        v = _buf[2 * r + u, pl.ds(L * j, L)]
                        _buf[2 * r + u, pl.ds(L * j, L)] = v * SCALE + pe_vs[j]
                return carry3

            lax.fori_loop(0, CHUNK // 2, row_body, 0)
            st[p] = pltpu.async_copy(
                bufs[p], out_hbm.at[s, pl.ds(c * CHUNK, CHUNK)], ssem[p])
        st[0].wait()
        st[1].wait()
        return carry

    lax.fori_loop(0, n_s, seq_body, 0)


def kernel(x, table, pe):
    idx_t = jnp.transpose(x).astype(jnp.int32).reshape(S, NCH, CHUNK)
    pe2 = pe[:S, 0, :]

    mesh = plsc.VectorSubcoreMesh(
        core_axis_name="c", subcore_axis_name="s",
        num_cores=NC, num_subcores=NS,
    )
    out = pl.kernel(
        _sc_body,
        out_type=jax.ShapeDtypeStruct((S, B, D), jnp.float32),
        mesh=mesh,
        scratch_types=[
            pltpu.VMEM((NCH, CHUNK), jnp.int32),
            pltpu.VMEM((CHUNK, D), jnp.float32),
            pltpu.VMEM((CHUNK, D), jnp.float32),
            pltpu.VMEM((D,), jnp.float32),
            pltpu.SemaphoreType.DMA,
            pltpu.SemaphoreType.DMA,
            pltpu.SemaphoreType.DMA,
            pltpu.SemaphoreType.DMA,
        ],
    )(table, idx_t, pe2)
    return out


# speedup vs baseline: 6.7881x; 1.1228x over previous
import math

import numpy as np

import jax
import jax.numpy as jnp
from jax import lax
from jax.experimental import pallas as pl
from jax.experimental.pallas import tpu as pltpu, tpu_sc as plsc

N_TOKENS = 100000
D = 128
B = 1024
S = 200

NC = 2
NS = 16
NW = NC * NS
L = 16

CHUNK = 128
NCH = B // CHUNK
NB = 4
SCALE = math.sqrt(float(D))

S_BASE = S // NW
S_EXTRA = S % NW
S_MAX = S_BASE + 1
S_PAD = 208

_STARTS = np.array([w * S_BASE + min(w, S_EXTRA) for w in range(NW)])
_WORKER_ROWS = _STARTS[:, None] + np.arange(S_MAX)[None, :]


def _sc_body(table_hbm, idx_hbm, pe_hbm, out_hbm,
             idx_all, pe_all, r0, r1, r2, r3,
             g0, g1, g2, g3, s0, s1, s2, s3):
    wid = lax.axis_index("s") * NC + lax.axis_index("c")
    n_s = S_BASE + (wid < S_EXTRA).astype(jnp.int32)
    start = wid * S_BASE + jnp.minimum(wid, S_EXTRA)

    bufs = (r0, r1, r2, r3)
    gsem = (g0, g1, g2, g3)
    ssem = (s0, s1, s2, s3)

    pltpu.sync_copy(idx_hbm.at[wid], idx_all)
    pltpu.sync_copy(pe_hbm.at[wid], pe_all)

    def seq_body(li, carry):
        s = start + li
        pe_vs = [pe_all[li, pl.ds(L * j, L)] for j in range(D // L)]

        g = [None] * NB
        st = [None] * NB
        for c in range(2):
            g[c] = pltpu.async_copy(table_hbm.at[idx_all.at[li, c]],
                                    bufs[c], gsem[c])
        for c in range(NCH):
            b = c % NB
            g[b].wait()

            def row_body(r, carry3, _buf=bufs[b]):
                for u in range(2):
                    for j in range(D // L):
                        v = _buf[2 * r + u, pl.ds(L * j, L)]
                        _buf[2 * r + u, pl.ds(L * j, L)] = v * SCALE + pe_vs[j]
                return carry3

            lax.fori_loop(0, CHUNK // 2, row_body, 0)
            st[b] = pltpu.async_copy(
                bufs[b], out_hbm.at[s, pl.ds(c * CHUNK, CHUNK)], ssem[b])
            if c + 2 < NCH:
                b2 = (c + 2) % NB
                if st[b2] is not None:
                    st[b2].wait()
                g[b2] = pltpu.async_copy(table_hbm.at[idx_all.at[li, c + 2]],
                                         bufs[b2], gsem[b2])
        for b in range(NB):
            st[b].wait()
        return carry

    lax.fori_loop(0, n_s, seq_body, 0)


def kernel(x, table, pe):
    idx_t = jnp.transpose(x).astype(jnp.int32).reshape(S, NCH, CHUNK)
    idx_t = jnp.pad(idx_t, ((0, S_PAD - S), (0, 0), (0, 0)))
    pe2 = jnp.pad(pe[:S, 0, :], ((0, S_PAD - S), (0, 0)))
    rows = _WORKER_ROWS
    idx_w = idx_t[rows]
    pe_w = pe2[rows]

    mesh = plsc.VectorSubcoreMesh(
        core_axis_name="c", subcore_axis_name="s",
        num_cores=NC, num_subcores=NS,
    )
    out = pl.kernel(
        _sc_body,
        out_type=jax.ShapeDtypeStruct((S, B, D), jnp.float32),
        mesh=mesh,
        scratch_types=(
            [pltpu.VMEM((S_MAX, NCH, CHUNK), jnp.int32),
             pltpu.VMEM((S_MAX, D), jnp.float32)]
            + [pltpu.VMEM((CHUNK, D), jnp.float32)] * NB
            + [pltpu.SemaphoreType.DMA] * (2 * NB)
        ),
    )(table, idx_w, pe_w)
    return out


# speedup vs baseline: 6.9209x; 1.0196x over previous
import math

import numpy as np

import jax
import jax.numpy as jnp
from jax import lax
from jax.experimental import pallas as pl
from jax.experimental.pallas import tpu as pltpu, tpu_sc as plsc

N_TOKENS = 100000
D = 128
B = 1024
S = 200

NC = 2
NS = 16
NW = NC * NS
L = 16

CHUNK = 128
NCH = B // CHUNK
TCH = S * NCH
WCH = TCH // NW
NB = 6
GL = 3
SCALE = math.sqrt(float(D))

S_MAX = 7
ROWS_W = S_MAX * NCH

_BASE_S = [(WCH * w) // NCH for w in range(NW)]
_ROW_MAP = np.minimum(
    np.array(_BASE_S)[:, None] * NCH + np.arange(ROWS_W)[None, :], TCH - 1)
_PE_MAP = np.minimum(np.array(_BASE_S)[:, None] + np.arange(S_MAX)[None, :],
                     S - 1)


def _sc_body(table_hbm, idx_hbm, pe_hbm, out_hbm,
             idx_all, pe_all, r0, r1, r2, r3, r4, r5,
             g0, g1, g2, g3, g4, g5, s0, s1, s2, s3, s4, s5):
    wid = lax.axis_index("s") * NC + lax.axis_index("c")
    base_s = (WCH * wid) // NCH
    r = (WCH * wid) % NCH

    bufs = (r0, r1, r2, r3, r4, r5)
    gsem = (g0, g1, g2, g3, g4, g5)
    ssem = (s0, s1, s2, s3, s4, s5)

    pltpu.sync_copy(idx_hbm.at[wid], idx_all)
    pltpu.sync_copy(pe_hbm.at[wid], pe_all)

    g = [None] * NB
    st = [None] * NB
    for t in range(GL):
        g[t] = pltpu.async_copy(table_hbm.at[idx_all.at[r + t]],
                                bufs[t], gsem[t])
    for t in range(WCH):
        b = t % NB
        row = r + t
        li = row // NCH
        s = base_s + li
        c = row % NCH
        g[b].wait()
        pe_vs = [pe_all[li, pl.ds(L * j, L)] for j in range(D // L)]

        def row_body(q, carry3, _buf=bufs[b], _pe=pe_vs):
            for u in range(2):
                for j in range(D // L):
                    v = _buf[2 * q + u, pl.ds(L * j, L)]
                    _buf[2 * q + u, pl.ds(L * j, L)] = v * SCALE + _pe[j]
            return carry3

        lax.fori_loop(0, CHUNK // 2, row_body, 0)
        st[b] = pltpu.async_copy(
            bufs[b],
            out_hbm.at[s, pl.ds(pl.multiple_of(c * CHUNK, CHUNK), CHUNK)],
            ssem[b])
        if t + GL < WCH:
            b3 = (t + GL) % NB
            if st[b3] is not None:
                st[b3].wait()
            g[b3] = pltpu.async_copy(table_hbm.at[idx_all.at[r + t + GL]],
                                     bufs[b3], gsem[b3])
    for b in range(NB):
        st[b].wait()


def kernel(x, table, pe):
    idx_flat = jnp.transpose(x).astype(jnp.int32).reshape(TCH, CHUNK)
    idx_w = idx_flat[_ROW_MAP]
    pe_w = pe[_PE_MAP, 0, :]

    mesh = plsc.VectorSubcoreMesh(
        core_axis_name="c", subcore_axis_name="s",
        num_cores=NC, num_subcores=NS,
    )
    out = pl.kernel(
        _sc_body,
        out_type=jax.ShapeDtypeStruct((S, B, D), jnp.float32),
        mesh=mesh,
        scratch_types=(
            [pltpu.VMEM((ROWS_W, CHUNK), jnp.int32),
             pltpu.VMEM((S_MAX, D), jnp.float32)]
            + [pltpu.VMEM((CHUNK, D), jnp.float32)] * NB
            + [pltpu.SemaphoreType.DMA] * (2 * NB)
        ),
    )(table, idx_w, pe_w)
    return out
